# baseline (speedup 1.0000x reference)
import functools

import jax
import jax.numpy as jnp
from jax import lax
from jax.experimental import pallas as pl
from jax.experimental.pallas import tpu as pltpu
from jax.experimental.pallas import tpu_sc as plsc

L_TOKENS = 16384
EMBED = 64
NC = 2
NS = 16
NW = NC * NS
CHUNK = 128
N_CHUNKS = L_TOKENS // (NW * CHUNK)

_sc_mesh = plsc.VectorSubcoreMesh(core_axis_name="c", subcore_axis_name="s")


@functools.partial(
    pl.kernel,
    out_type=jax.ShapeDtypeStruct((NW, EMBED), jnp.float32),
    mesh=_sc_mesh,
    compiler_params=pltpu.CompilerParams(use_tc_tiling_on_sc=False),
    scratch_types=[
        pltpu.VMEM((N_CHUNKS, CHUNK), jnp.int32),
        pltpu.VMEM((2, CHUNK, EMBED), jnp.float32),
        pltpu.VMEM((EMBED,), jnp.float32),
        pltpu.SemaphoreType.DMA,
        pltpu.SemaphoreType.DMA,
    ],
)
def _gather_sum(idx_hbm, table_hbm, out_hbm, idx_v, rows_v, acc_v, sem0, sem1):
    wid = lax.axis_index("s") * NC + lax.axis_index("c")
    pltpu.sync_copy(idx_hbm.at[pl.ds(wid * N_CHUNKS, N_CHUNKS)], idx_v)

    sems = (sem0, sem1)
    pltpu.async_copy(table_hbm.at[idx_v.at[0]], rows_v.at[0], sems[0])

    zero = jnp.zeros((16,), jnp.float32)
    accs = (zero, zero, zero, zero)

    def accum_body(rows_ref):
        def body(i, a):
            a0, a1, a2, a3 = a
            a0 = a0 + rows_ref[i, pl.ds(0, 16)]
            a1 = a1 + rows_ref[i, pl.ds(16, 16)]
            a2 = a2 + rows_ref[i, pl.ds(32, 16)]
            a3 = a3 + rows_ref[i, pl.ds(48, 16)]
            return (a0, a1, a2, a3)
        return body

    for j in range(N_CHUNKS):
        buf = j % 2
        pltpu.make_async_copy(
            table_hbm.at[idx_v.at[j]], rows_v.at[buf], sems[buf]
        ).wait()
        if j + 1 < N_CHUNKS:
            pltpu.async_copy(
                table_hbm.at[idx_v.at[j + 1]], rows_v.at[1 - buf], sems[1 - buf]
            )
        accs = lax.fori_loop(0, CHUNK, accum_body(rows_v.at[buf]), accs)

    acc_v[pl.ds(0, 16)] = accs[0]
    acc_v[pl.ds(16, 16)] = accs[1]
    acc_v[pl.ds(32, 16)] = accs[2]
    acc_v[pl.ds(48, 16)] = accs[3]
    pltpu.sync_copy(acc_v, out_hbm.at[wid])


def _mlp_body(p_ref, w1_ref, b1_ref, w2_ref, b2_ref, o_ref):
    e = jnp.sum(p_ref[...], axis=0, keepdims=True)
    h = lax.dot_general(e, w1_ref[...], (((1,), (1,)), ((), ())),
                        preferred_element_type=jnp.float32)
    h = jnp.maximum(h + b1_ref[...], 0.0)
    o = lax.dot_general(h, w2_ref[...], (((1,), (1,)), ((), ())),
                        preferred_element_type=jnp.float32)
    o_ref[...] = jnp.maximum(o + b2_ref[...], 0.0)


def kernel(inputs, table, W1, b1, W2, b2):
    idx = inputs.astype(jnp.int32).reshape(NW * N_CHUNKS, CHUNK)
    partials = _gather_sum(idx, table)
    out = pl.pallas_call(
        _mlp_body,
        out_shape=jax.ShapeDtypeStruct((1, W2.shape[0]), jnp.float32),
    )(partials, W1, b1.reshape(1, -1), W2, b2.reshape(1, -1))
    return out


# speedup vs baseline: 1.6824x; 1.6824x over previous
import functools

import jax
import jax.numpy as jnp
from jax import lax
from jax.experimental import pallas as pl
from jax.experimental.pallas import tpu as pltpu
from jax.experimental.pallas import tpu_sc as plsc

L_TOKENS = 16384
EMBED = 64
NC = 2
NS = 16
NW = NC * NS
PER_W = L_TOKENS // NW
BATCH = 16
NBATCH = PER_W // BATCH
NB2 = NBATCH // 2

_sc_mesh = plsc.VectorSubcoreMesh(core_axis_name="c", subcore_axis_name="s")


@functools.partial(
    pl.kernel,
    out_type=jax.ShapeDtypeStruct((NW, EMBED), jnp.float32),
    mesh=_sc_mesh,
    scratch_types=[
        pltpu.VMEM((PER_W,), jnp.int32),
        pltpu.VMEM((2, BATCH, EMBED), jnp.float32),
        pltpu.VMEM((EMBED,), jnp.float32),
        pltpu.SemaphoreType.DMA,
        pltpu.SemaphoreType.DMA,
    ],
)
def _gather_sum(idx_hbm, table_hbm, out_hbm, idx_v, rows_v, acc_v,
                sem0, sem1):
    wid = lax.axis_index("s") * NC + lax.axis_index("c")
    base = wid * PER_W
    pltpu.sync_copy(idx_hbm.at[pl.ds(base, PER_W)], idx_v)

    def fire(b, buf, sem):
        v = idx_v[pl.ds(b * BATCH, BATCH)]
        dst = rows_v.at[buf]
        for lane in range(BATCH):
            r = v[lane]
            pltpu.async_copy(
                table_hbm.at[pl.ds(r, 1)], dst.at[pl.ds(lane, 1)], sem
            )

    def drain(buf, sem):
        pltpu.make_async_copy(
            table_hbm.at[pl.ds(0, BATCH)], rows_v.at[buf], sem
        ).wait()

    def accum(buf, accs):
        rows_ref = rows_v.at[buf]

        def body(i, a):
            a0, a1, a2, a3 = a
            a0 = a0 + rows_ref[i, pl.ds(0, 16)]
            a1 = a1 + rows_ref[i, pl.ds(16, 16)]
            a2 = a2 + rows_ref[i, pl.ds(32, 16)]
            a3 = a3 + rows_ref[i, pl.ds(48, 16)]
            return (a0, a1, a2, a3)

        return lax.fori_loop(0, BATCH, body, accs)

    fire(0, 0, sem0)
    zero = jnp.zeros((16,), jnp.float32)

    def outer(gg, accs):
        b0 = 2 * gg
        fire(b0 + 1, 1, sem1)
        drain(0, sem0)
        accs = accum(0, accs)

        @pl.when(gg + 1 < NB2)
        def _():
            fire(b0 + 2, 0, sem0)

        drain(1, sem1)
        accs = accum(1, accs)
        return accs

    accs = lax.fori_loop(0, NB2, outer, (zero, zero, zero, zero))

    acc_v[pl.ds(0, 16)] = accs[0]
    acc_v[pl.ds(16, 16)] = accs[1]
    acc_v[pl.ds(32, 16)] = accs[2]
    acc_v[pl.ds(48, 16)] = accs[3]
    pltpu.sync_copy(acc_v, out_hbm.at[wid])


def _mlp_body(p_ref, w1_ref, b1_ref, w2_ref, b2_ref, o_ref):
    e = jnp.sum(p_ref[...], axis=0, keepdims=True)
    h = lax.dot_general(e, w1_ref[...], (((1,), (1,)), ((), ())),
                        preferred_element_type=jnp.float32)
    h = jnp.maximum(h + b1_ref[...], 0.0)
    o = lax.dot_general(h, w2_ref[...], (((1,), (1,)), ((), ())),
                        preferred_element_type=jnp.float32)
    o_ref[...] = jnp.maximum(o + b2_ref[...], 0.0)


def kernel(inputs, table, W1, b1, W2, b2):
    idx = inputs.astype(jnp.int32)
    partials = _gather_sum(idx, table)
    out = pl.pallas_call(
        _mlp_body,
        out_shape=jax.ShapeDtypeStruct((1, W2.shape[0]), jnp.float32),
    )(partials, W1, b1.reshape(1, -1), W2, b2.reshape(1, -1))
    return out


# speedup vs baseline: 3.8673x; 2.2987x over previous
import functools

import jax
import jax.numpy as jnp
from jax import lax
from jax.experimental import pallas as pl
from jax.experimental.pallas import tpu as pltpu
from jax.experimental.pallas import tpu_sc as plsc

VOCAB = 1000000
EMBED = 64
L_TOKENS = 16384
NC = 2
NS = 16
NW = NC * NS
NBINS = 1 << 20
BINS_PER_W = NBINS // NW
CROWS = 128
CCOLS = NBINS // CROWS
ROWS_PER_W = CROWS // NW

_sc_mesh = plsc.VectorSubcoreMesh(core_axis_name="c", subcore_axis_name="s")


@functools.partial(
    pl.kernel,
    out_type=jax.ShapeDtypeStruct((NBINS,), jnp.float32),
    mesh=_sc_mesh,
    compiler_params=pltpu.CompilerParams(needs_layout_passes=False),
    scratch_types=[
        pltpu.VMEM((L_TOKENS,), jnp.int32),
        pltpu.VMEM((BINS_PER_W,), jnp.float32),
    ],
)
def _histogram(idx_hbm, zeros_hbm, out_hbm, idx_v, hist_v):
    wid = lax.axis_index("s") * NC + lax.axis_index("c")
    lo = wid * BINS_PER_W
    pltpu.sync_copy(zeros_hbm, hist_v)
    pltpu.sync_copy(idx_hbm, idx_v)

    ones = jnp.ones((16,), jnp.float32)

    def body(k, carry):
        v = idx_v[pl.ds(k * 16, 16)]
        vl = v - lo
        m = (vl >= 0) & (vl < BINS_PER_W)
        vl = jnp.where(m, vl, 0)
        plsc.addupdate_scatter(hist_v, [vl], ones, mask=m)
        return carry

    lax.fori_loop(0, L_TOKENS // 16, body, 0)

    pltpu.sync_copy(hist_v, out_hbm.at[pl.ds(lo, BINS_PER_W)])


BLK = 8192
NFULL = VOCAB // BLK
NPAIR = NFULL // 2
ALIGNED = NFULL * BLK
TAIL = VOCAB - ALIGNED
TAIL_A = 512
RAG = TAIL - TAIL_A


def _matvec_mlp_body(t_hbm, c_hbm, rag_t_ref, rag_c_ref, w1_ref, b1_ref,
                     w2_ref, b2_ref, o_ref,
                     tbuf, cbuf, acc_ref, tsem0, tsem1, csem0, csem1):
    tsems = (tsem0, tsem1)
    csems = (csem0, csem1)

    def fire(b, buf):
        pltpu.async_copy(
            t_hbm.at[:, pl.ds(b * BLK, BLK)], tbuf.at[buf], tsems[buf])
        pltpu.async_copy(
            c_hbm.at[pl.ds(b * BLK, BLK)], cbuf.at[buf], csems[buf])

    def fire_t(b, buf):
        pltpu.async_copy(
            t_hbm.at[:, pl.ds(b * BLK, BLK)], tbuf.at[buf], tsems[buf])
        pltpu.async_copy(
            c_hbm.at[pl.ds(b * BLK, BLK)], cbuf.at[buf], csems[buf])

    def wait(buf):
        pltpu.make_async_copy(
            t_hbm.at[:, pl.ds(0, BLK)], tbuf.at[buf], tsems[buf]).wait()
        pltpu.make_async_copy(
            c_hbm.at[pl.ds(0, BLK)], cbuf.at[buf], csems[buf]).wait()

    def accum(buf):
        acc_ref[...] += tbuf[buf] * cbuf[buf]

    acc_ref[...] = jnp.zeros_like(acc_ref)
    fire(0, 0)

    def pair(gg, carry):
        b0 = 2 * gg
        fire_t(b0 + 1, 1)
        wait(0)
        accum(0)

        @pl.when(gg + 1 < NPAIR)
        def _():
            fire_t(b0 + 2, 0)

        wait(1)
        accum(1)
        return carry

    lax.fori_loop(0, NPAIR, pair, 0)

    pltpu.async_copy(
        t_hbm.at[:, pl.ds(ALIGNED, TAIL_A)],
        tbuf.at[0].at[:, pl.ds(0, TAIL_A)], tsems[0])
    pltpu.async_copy(
        c_hbm.at[pl.ds(ALIGNED, TAIL_A)],
        cbuf.at[0].at[pl.ds(0, TAIL_A)], csems[0])
    pltpu.make_async_copy(
        t_hbm.at[:, pl.ds(0, TAIL_A)],
        tbuf.at[0].at[:, pl.ds(0, TAIL_A)], tsems[0]).wait()
    pltpu.make_async_copy(
        c_hbm.at[pl.ds(0, TAIL_A)],
        cbuf.at[0].at[pl.ds(0, TAIL_A)], csems[0]).wait()
    col = jax.lax.broadcasted_iota(jnp.int32, (1, BLK), 1)
    acc_ref[...] += jnp.where(col < TAIL_A, tbuf[0] * cbuf[0], 0.0)

    e = jnp.sum(acc_ref[...], axis=1, keepdims=True)
    e = e + lax.dot_general(rag_t_ref[...], rag_c_ref[...],
                            (((1,), (0,)), ((), ())),
                            preferred_element_type=jnp.float32)
    h = lax.dot_general(w1_ref[...], e, (((1,), (0,)), ((), ())),
                        preferred_element_type=jnp.float32)
    h = jnp.maximum(h + b1_ref[...], 0.0)
    o = lax.dot_general(h, w2_ref[...], (((0,), (1,)), ((), ())),
                        preferred_element_type=jnp.float32)
    o_ref[...] = jnp.maximum(o + b2_ref[...], 0.0)


def kernel(inputs, table, W1, b1, W2, b2):
    idx = inputs.astype(jnp.int32)
    zeros = jnp.zeros((BINS_PER_W,), jnp.float32)
    counts = _histogram(idx, zeros)
    tableT = table.T
    out = pl.pallas_call(
        _matvec_mlp_body,
        in_specs=[
            pl.BlockSpec(memory_space=pltpu.HBM),
            pl.BlockSpec(memory_space=pltpu.HBM),
            pl.BlockSpec((EMBED, RAG), lambda: (0, 0)),
            pl.BlockSpec((RAG, 1), lambda: (0, 0)),
            pl.BlockSpec((128, EMBED), lambda: (0, 0)),
            pl.BlockSpec((128, 1), lambda: (0, 0)),
            pl.BlockSpec((2048, 128), lambda: (0, 0)),
            pl.BlockSpec((1, 2048), lambda: (0, 0)),
        ],
        out_specs=pl.BlockSpec((1, 2048), lambda: (0, 0)),
        out_shape=jax.ShapeDtypeStruct((1, 2048), jnp.float32),
        scratch_shapes=[
            pltpu.VMEM((2, EMBED, BLK), jnp.float32),
            pltpu.VMEM((2, BLK), jnp.float32),
            pltpu.VMEM((EMBED, BLK), jnp.float32),
            pltpu.SemaphoreType.DMA,
            pltpu.SemaphoreType.DMA,
            pltpu.SemaphoreType.DMA,
            pltpu.SemaphoreType.DMA,
        ],
    )(tableT, counts,
      tableT[:, ALIGNED + TAIL_A:].astype(jnp.float32),
      counts[ALIGNED + TAIL_A:ALIGNED + TAIL_A + RAG].reshape(RAG, 1),
      W1, b1.reshape(128, 1), W2, b2.reshape(1, 2048))
    return out


# speedup vs baseline: 4.9760x; 1.2867x over previous
import functools

import jax
import jax.numpy as jnp
from jax import lax
from jax.experimental import pallas as pl
from jax.experimental.pallas import tpu as pltpu
from jax.experimental.pallas import tpu_sc as plsc

VOCAB = 1000000
EMBED = 64
L_TOKENS = 16384
NC = 2
NS = 16
NW = NC * NS
NBINS = 1 << 20
BINS_PER_W = NBINS // NW
CROWS = 128
CCOLS = NBINS // CROWS
ROWS_PER_W = CROWS // NW

_sc_mesh = plsc.VectorSubcoreMesh(core_axis_name="c", subcore_axis_name="s")


@functools.partial(
    pl.kernel,
    out_type=jax.ShapeDtypeStruct((NBINS,), jnp.float32),
    mesh=_sc_mesh,
    compiler_params=pltpu.CompilerParams(needs_layout_passes=False),
    scratch_types=[
        pltpu.VMEM((L_TOKENS,), jnp.int32),
        pltpu.VMEM((BINS_PER_W,), jnp.float32),
    ],
)
def _histogram(idx_hbm, zeros_hbm, out_hbm, idx_v, hist_v):
    wid = lax.axis_index("s") * NC + lax.axis_index("c")
    lo = wid * BINS_PER_W
    pltpu.sync_copy(zeros_hbm, hist_v)
    pltpu.sync_copy(idx_hbm, idx_v)

    ones = jnp.ones((16,), jnp.float32)

    def body(k, carry):
        for u in range(4):
            v = idx_v[pl.ds((k * 4 + u) * 16, 16)]
            vl = v - lo
            m = (vl >= 0) & (vl < BINS_PER_W)
            vl = jnp.where(m, vl, 0)
            plsc.addupdate_scatter(hist_v, [vl], ones, mask=m)
        return carry

    lax.fori_loop(0, L_TOKENS // 64, body, 0)

    pltpu.sync_copy(hist_v, out_hbm.at[pl.ds(lo, BINS_PER_W)])


BLK = 16384
ACCW = 2048
NFULL = VOCAB // BLK
NPAIR = NFULL // 2
ALIGNED = NFULL * BLK
TAIL = VOCAB - ALIGNED
TAIL_A = 512
RAG = TAIL - TAIL_A


def _matvec_mlp_body(t_hbm, c_hbm, rag_t_ref, rag_c_ref, w1_ref, b1_ref,
                     w2_ref, b2_ref, o_ref,
                     tbuf, cbuf, acc_ref, tsem0, tsem1, csem0, csem1):
    tsems = (tsem0, tsem1)
    csems = (csem0, csem1)

    def fire(b, buf):
        pltpu.async_copy(
            t_hbm.at[:, pl.ds(b * BLK, BLK)], tbuf.at[buf], tsems[buf])
        pltpu.async_copy(
            c_hbm.at[pl.ds(b * BLK, BLK)], cbuf.at[buf], csems[buf])

    def fire_t(b, buf):
        pltpu.async_copy(
            t_hbm.at[:, pl.ds(b * BLK, BLK)], tbuf.at[buf], tsems[buf])
        pltpu.async_copy(
            c_hbm.at[pl.ds(b * BLK, BLK)], cbuf.at[buf], csems[buf])

    def wait(buf):
        pltpu.make_async_copy(
            t_hbm.at[:, pl.ds(0, BLK)], tbuf.at[buf], tsems[buf]).wait()
        pltpu.make_async_copy(
            c_hbm.at[pl.ds(0, BLK)], cbuf.at[buf], csems[buf]).wait()

    def accum(buf):
        t = tbuf[buf] * cbuf[buf]
        a = t[:, :8192] + t[:, 8192:]
        b = a[:, :4096] + a[:, 4096:]
        acc_ref[...] += b[:, :ACCW] + b[:, ACCW:]

    def accum_masked(buf):
        col = jax.lax.broadcasted_iota(jnp.int32, (1, BLK), 1)
        t = jnp.where(col < TAIL_A, tbuf[buf] * cbuf[buf], 0.0)
        a = t[:, :8192] + t[:, 8192:]
        b = a[:, :4096] + a[:, 4096:]
        acc_ref[...] += b[:, :ACCW] + b[:, ACCW:]

    acc_ref[...] = jnp.zeros_like(acc_ref)
    fire(0, 0)

    def pair(gg, carry):
        b0 = 2 * gg
        fire_t(b0 + 1, 1)
        wait(0)
        accum(0)

        @pl.when(b0 + 2 < NFULL)
        def _():
            fire_t(b0 + 2, 0)

        wait(1)
        accum(1)
        return carry

    lax.fori_loop(0, NPAIR, pair, 0)
    wait(0)
    accum(0)

    pltpu.async_copy(
        t_hbm.at[:, pl.ds(ALIGNED, TAIL_A)],
        tbuf.at[1].at[:, pl.ds(0, TAIL_A)], tsems[1])
    pltpu.async_copy(
        c_hbm.at[pl.ds(ALIGNED, TAIL_A)],
        cbuf.at[1].at[pl.ds(0, TAIL_A)], csems[1])
    pltpu.make_async_copy(
        t_hbm.at[:, pl.ds(0, TAIL_A)],
        tbuf.at[1].at[:, pl.ds(0, TAIL_A)], tsems[1]).wait()
    pltpu.make_async_copy(
        c_hbm.at[pl.ds(0, TAIL_A)],
        cbuf.at[1].at[pl.ds(0, TAIL_A)], csems[1]).wait()
    accum_masked(1)

    e = jnp.sum(acc_ref[...], axis=1, keepdims=True)
    e = e + lax.dot_general(rag_t_ref[...], rag_c_ref[...],
                            (((1,), (0,)), ((), ())),
                            preferred_element_type=jnp.float32)
    h = lax.dot_general(w1_ref[...], e, (((1,), (0,)), ((), ())),
                        preferred_element_type=jnp.float32)
    h = jnp.maximum(h + b1_ref[...], 0.0)
    o = lax.dot_general(h, w2_ref[...], (((0,), (1,)), ((), ())),
                        preferred_element_type=jnp.float32)
    o_ref[...] = jnp.maximum(o + b2_ref[...], 0.0)


def kernel(inputs, table, W1, b1, W2, b2):
    idx = inputs.astype(jnp.int32)
    zeros = jnp.zeros((BINS_PER_W,), jnp.float32)
    counts = _histogram(idx, zeros)
    tableT = table.T
    out = pl.pallas_call(
        _matvec_mlp_body,
        in_specs=[
            pl.BlockSpec(memory_space=pltpu.HBM),
            pl.BlockSpec(memory_space=pltpu.HBM),
            pl.BlockSpec((EMBED, RAG), lambda: (0, 0)),
            pl.BlockSpec((RAG, 1), lambda: (0, 0)),
            pl.BlockSpec((128, EMBED), lambda: (0, 0)),
            pl.BlockSpec((128, 1), lambda: (0, 0)),
            pl.BlockSpec((2048, 128), lambda: (0, 0)),
            pl.BlockSpec((1, 2048), lambda: (0, 0)),
        ],
        out_specs=pl.BlockSpec((1, 2048), lambda: (0, 0)),
        out_shape=jax.ShapeDtypeStruct((1, 2048), jnp.float32),
        scratch_shapes=[
            pltpu.VMEM((2, EMBED, BLK), jnp.float32),
            pltpu.VMEM((2, BLK), jnp.float32),
            pltpu.VMEM((EMBED, ACCW), jnp.float32),
            pltpu.SemaphoreType.DMA,
            pltpu.SemaphoreType.DMA,
            pltpu.SemaphoreType.DMA,
            pltpu.SemaphoreType.DMA,
        ],
    )(tableT, counts,
      tableT[:, ALIGNED + TAIL_A:].astype(jnp.float32),
      counts[ALIGNED + TAIL_A:ALIGNED + TAIL_A + RAG].reshape(RAG, 1),
      W1, b1.reshape(128, 1), W2, b2.reshape(1, 2048))
    return out


# speedup vs baseline: 5.3055x; 1.0662x over previous
import functools

import jax
import jax.numpy as jnp
from jax import lax
from jax.experimental import pallas as pl
from jax.experimental.pallas import tpu as pltpu
from jax.experimental.pallas import tpu_sc as plsc

VOCAB = 1000000
EMBED = 64
L_TOKENS = 16384
NC = 2
NS = 16
NW = NC * NS
NBINS = 1 << 20
BINS_PER_W = NBINS // NW
CROWS = 128
CCOLS = NBINS // CROWS
ROWS_PER_W = CROWS // NW

_sc_mesh = plsc.VectorSubcoreMesh(core_axis_name="c", subcore_axis_name="s")


@functools.partial(
    pl.kernel,
    out_type=jax.ShapeDtypeStruct((NBINS,), jnp.float32),
    mesh=_sc_mesh,
    compiler_params=pltpu.CompilerParams(needs_layout_passes=False),
    scratch_types=[
        pltpu.VMEM((L_TOKENS,), jnp.int32),
        pltpu.VMEM((BINS_PER_W,), jnp.float32),
    ],
)
def _histogram(idx_hbm, zeros_hbm, out_hbm, idx_v, hist_v):
    wid = lax.axis_index("s") * NC + lax.axis_index("c")
    lo = wid * BINS_PER_W
    pltpu.sync_copy(zeros_hbm, hist_v)
    pltpu.sync_copy(idx_hbm, idx_v)

    ones = jnp.ones((16,), jnp.float32)

    def body(k, carry):
        for u in range(4):
            v = idx_v[pl.ds((k * 4 + u) * 16, 16)]
            vl = v - lo
            m = (vl >= 0) & (vl < BINS_PER_W)
            vl = jnp.where(m, vl, 0)
            plsc.addupdate_scatter(hist_v, [vl], ones, mask=m)
        return carry

    lax.fori_loop(0, L_TOKENS // 64, body, 0)

    pltpu.sync_copy(hist_v, out_hbm.at[pl.ds(lo, BINS_PER_W)])


BLK = 16384
ACCW = 2048
NFULL = VOCAB // BLK
NPAIR = NFULL // 2
ALIGNED = NFULL * BLK
TAIL = VOCAB - ALIGNED
TAIL_A = 512
RAG = TAIL - TAIL_A


NBUF = 4
NGRP = NFULL // NBUF


def _matvec_mlp_body(t_hbm, c_hbm, rag_t_ref, rag_c_ref, w1_ref, b1_ref,
                     w2_ref, b2_ref, o_ref,
                     tbuf, cbuf, acc_ref, tsem0, tsem1, tsem2, tsem3,
                     csem0, csem1, csem2, csem3):
    tsems = (tsem0, tsem1, tsem2, tsem3)
    csems = (csem0, csem1, csem2, csem3)

    def fire(b, buf):
        pltpu.async_copy(
            t_hbm.at[:, pl.ds(b * BLK, BLK)], tbuf.at[buf], tsems[buf])
        pltpu.async_copy(
            c_hbm.at[pl.ds(b * BLK, BLK)], cbuf.at[buf], csems[buf])

    def fire_t(b, buf):
        pltpu.async_copy(
            t_hbm.at[:, pl.ds(b * BLK, BLK)], tbuf.at[buf], tsems[buf])
        pltpu.async_copy(
            c_hbm.at[pl.ds(b * BLK, BLK)], cbuf.at[buf], csems[buf])

    def wait(buf):
        pltpu.make_async_copy(
            t_hbm.at[:, pl.ds(0, BLK)], tbuf.at[buf], tsems[buf]).wait()
        pltpu.make_async_copy(
            c_hbm.at[pl.ds(0, BLK)], cbuf.at[buf], csems[buf]).wait()

    def accum(buf):
        t = tbuf[buf] * cbuf[buf]
        a = t[:, :8192] + t[:, 8192:]
        b = a[:, :4096] + a[:, 4096:]
        acc_ref[...] += b[:, :ACCW] + b[:, ACCW:]

    def accum_masked(buf):
        col = jax.lax.broadcasted_iota(jnp.int32, (1, BLK), 1)
        t = jnp.where(col < TAIL_A, tbuf[buf] * cbuf[buf], 0.0)
        a = t[:, :8192] + t[:, 8192:]
        b = a[:, :4096] + a[:, 4096:]
        acc_ref[...] += b[:, :ACCW] + b[:, ACCW:]

    acc_ref[...] = jnp.zeros_like(acc_ref)
    fire(0, 0)
    fire(1, 1)
    fire(2, 2)

    def grp(gg, carry):
        b0 = NBUF * gg
        for j in range(NBUF):
            b = b0 + j

            @pl.when(b + 3 < NFULL)
            def _(b=b, j=j):
                fire_t(b + 3, (j + 3) % NBUF)

            wait(j)
            accum(j)
        return carry

    lax.fori_loop(0, NGRP, grp, 0)
    wait(0)
    accum(0)

    pltpu.async_copy(
        t_hbm.at[:, pl.ds(ALIGNED, TAIL_A)],
        tbuf.at[1].at[:, pl.ds(0, TAIL_A)], tsems[1])
    pltpu.async_copy(
        c_hbm.at[pl.ds(ALIGNED, TAIL_A)],
        cbuf.at[1].at[pl.ds(0, TAIL_A)], csems[1])
    pltpu.make_async_copy(
        t_hbm.at[:, pl.ds(0, TAIL_A)],
        tbuf.at[1].at[:, pl.ds(0, TAIL_A)], tsems[1]).wait()
    pltpu.make_async_copy(
        c_hbm.at[pl.ds(0, TAIL_A)],
        cbuf.at[1].at[pl.ds(0, TAIL_A)], csems[1]).wait()
    accum_masked(1)

    e = jnp.sum(acc_ref[...], axis=1, keepdims=True)
    e = e + lax.dot_general(rag_t_ref[...], rag_c_ref[...],
                            (((1,), (0,)), ((), ())),
                            preferred_element_type=jnp.float32)
    h = lax.dot_general(w1_ref[...], e, (((1,), (0,)), ((), ())),
                        preferred_element_type=jnp.float32)
    h = jnp.maximum(h + b1_ref[...], 0.0)
    o = lax.dot_general(h, w2_ref[...], (((0,), (1,)), ((), ())),
                        preferred_element_type=jnp.float32)
    o_ref[...] = jnp.maximum(o + b2_ref[...], 0.0)


def kernel(inputs, table, W1, b1, W2, b2):
    idx = inputs.astype(jnp.int32)
    zeros = jnp.zeros((BINS_PER_W,), jnp.float32)
    counts = _histogram(idx, zeros)
    tableT = table.T
    out = pl.pallas_call(
        _matvec_mlp_body,
        in_specs=[
            pl.BlockSpec(memory_space=pltpu.HBM),
            pl.BlockSpec(memory_space=pltpu.HBM),
            pl.BlockSpec((EMBED, RAG), lambda: (0, 0)),
            pl.BlockSpec((RAG, 1), lambda: (0, 0)),
            pl.BlockSpec((128, EMBED), lambda: (0, 0)),
            pl.BlockSpec((128, 1), lambda: (0, 0)),
            pl.BlockSpec((2048, 128), lambda: (0, 0)),
            pl.BlockSpec((1, 2048), lambda: (0, 0)),
        ],
        out_specs=pl.BlockSpec((1, 2048), lambda: (0, 0)),
        out_shape=jax.ShapeDtypeStruct((1, 2048), jnp.float32),
        scratch_shapes=[
            pltpu.VMEM((NBUF, EMBED, BLK), jnp.float32),
            pltpu.VMEM((NBUF, BLK), jnp.float32),
            pltpu.VMEM((EMBED, ACCW), jnp.float32),
            pltpu.SemaphoreType.DMA,
            pltpu.SemaphoreType.DMA,
            pltpu.SemaphoreType.DMA,
            pltpu.SemaphoreType.DMA,
            pltpu.SemaphoreType.DMA,
            pltpu.SemaphoreType.DMA,
            pltpu.SemaphoreType.DMA,
            pltpu.SemaphoreType.DMA,
        ],
    )(tableT, counts,
      tableT[:, ALIGNED + TAIL_A:].astype(jnp.float32),
      counts[ALIGNED + TAIL_A:ALIGNED + TAIL_A + RAG].reshape(RAG, 1),
      W1, b1.reshape(128, 1), W2, b2.reshape(1, 2048))
    return out
